# Initial kernel scaffold; baseline (speedup 1.0000x reference)
import jax
import jax.numpy as jnp
from jax.experimental import pallas as pl


def kernel(trigram_ids, interacted_rate, trigram_table, subreddit_table, W1, b1, W2, b2, W3, b3):
    raise NotImplementedError("write your pallas kernel here")


# speedup vs baseline: 2.6609x; 2.6609x over previous
import functools

import jax
import jax.numpy as jnp
from jax import lax
from jax.experimental import pallas as pl
from jax.experimental.pallas import tpu as pltpu
from jax.experimental.pallas import tpu_sc as plsc

NC = 2
NS = 16
NW = NC * NS


def _sc_gather(table, idx2d, gathers_per_chunk):
    n_idx_rows, R = idx2d.shape
    N = n_idx_rows * R
    D = table.shape[1]
    per_w_rows = n_idx_rows // NW
    K = gathers_per_chunk
    n_chunks = per_w_rows // K
    assert per_w_rows * NW == n_idx_rows and n_chunks * K == per_w_rows

    mesh = plsc.VectorSubcoreMesh(core_axis_name="c", subcore_axis_name="s")

    @functools.partial(
        pl.kernel,
        mesh=mesh,
        out_type=jax.ShapeDtypeStruct((N, D), table.dtype),
        scratch_types=[
            pltpu.VMEM((K, R), jnp.int32),
            pltpu.VMEM((K * R, D), jnp.float32),
            pltpu.SemaphoreType.DMA,
        ],
    )
    def gather_kernel(table_hbm, idx_hbm, out_hbm, idx_v, rows_v, sem):
        wid = lax.axis_index("s") * NC + lax.axis_index("c")
        base = wid * per_w_rows

        @pl.loop(0, n_chunks)
        def _(ci):
            row0 = base + ci * K
            pltpu.sync_copy(idx_hbm.at[pl.ds(row0, K)], idx_v)
            copies = [
                pltpu.async_copy(
                    table_hbm.at[idx_v.at[j]],
                    rows_v.at[pl.ds(j * R, R)],
                    sem,
                )
                for j in range(K)
            ]
            for c in copies:
                c.wait()
            pltpu.sync_copy(rows_v, out_hbm.at[pl.ds(row0 * R, K * R)])

    return gather_kernel(table, idx2d)


def _tc_mlp(gt, gi, true_l, w1t, b1, w2at, w2bt, b2, w3t, b3):
    B, S, F = gt.shape
    L = true_l
    BB = 16

    def body(gt_ref, gi_ref, w1t_ref, b1_ref, w2at_ref, w2bt_ref, b2_ref,
             w3t_ref, b3_ref, o_ref):
        acc = gt_ref[:, 0, :]
        for s in range(1, S):
            acc = acc + gt_ref[:, s, :]
        x = acc * (1.0 / S)
        t = jnp.dot(x, w1t_ref[...], preferred_element_type=jnp.float32)
        t = jnp.maximum(t + b1_ref[...], 0.0)
        acc2 = gi_ref[:, 0, :]
        for s in range(1, L):
            acc2 = acc2 + gi_ref[:, s, :]
        y = acc2 * (1.0 / L)
        h = (jnp.dot(y, w2at_ref[...], preferred_element_type=jnp.float32)
             + jnp.dot(t, w2bt_ref[...], preferred_element_type=jnp.float32))
        h = jnp.maximum(h + b2_ref[...], 0.0)
        o_ref[...] = (jnp.dot(h, w3t_ref[...], preferred_element_type=jnp.float32)
                      + b3_ref[...])

    return pl.pallas_call(
        body,
        grid=(B // BB,),
        in_specs=[
            pl.BlockSpec((BB, S, F), lambda i: (i, 0, 0)),
            pl.BlockSpec((BB, gi.shape[1], gi.shape[2]), lambda i: (i, 0, 0)),
            pl.BlockSpec(w1t.shape, lambda i: (0, 0)),
            pl.BlockSpec(b1.shape, lambda i: (0, 0)),
            pl.BlockSpec(w2at.shape, lambda i: (0, 0)),
            pl.BlockSpec(w2bt.shape, lambda i: (0, 0)),
            pl.BlockSpec(b2.shape, lambda i: (0, 0)),
            pl.BlockSpec(w3t.shape, lambda i: (0, 0)),
            pl.BlockSpec(b3.shape, lambda i: (0, 0)),
        ],
        out_specs=pl.BlockSpec((BB, w3t.shape[1]), lambda i: (i, 0)),
        out_shape=jax.ShapeDtypeStruct((B, w3t.shape[1]), jnp.float32),
    )(gt, gi, w1t, b1, w2at, w2bt, b2, w3t, b3)


def kernel(trigram_ids, interacted_rate, trigram_table, subreddit_table,
           W1, b1, W2, b2, W3, b3):
    B, S, TRI = trigram_ids.shape
    L = interacted_rate.shape[1]
    EMB = trigram_table.shape[1]

    LP = 64
    tri_idx = trigram_ids.astype(jnp.int32).reshape(-1, 64)
    int_pad = jnp.pad(interacted_rate.astype(jnp.int32),
                      ((0, 0), (0, LP - L)))
    int_idx = int_pad.reshape(-1, 64)

    g_tri = _sc_gather(trigram_table, tri_idx, 8)
    g_int = _sc_gather(subreddit_table, int_idx, 8)

    gt = g_tri.reshape(B, S, TRI * EMB)
    gi = g_int.reshape(B, LP, EMB)

    return _tc_mlp(
        gt, gi, L,
        W1.T, b1.reshape(1, -1),
        W2[:, :EMB].T, W2[:, EMB:].T, b2.reshape(1, -1),
        W3.T, b3.reshape(1, -1),
    )


# speedup vs baseline: 2.7045x; 1.0164x over previous
import functools

import jax
import jax.numpy as jnp
from jax import lax
from jax.experimental import pallas as pl
from jax.experimental.pallas import tpu as pltpu
from jax.experimental.pallas import tpu_sc as plsc

NC = 2
NS = 16
NW = NC * NS


K = 4
R = 64
CH = K * R
SBR = 16


def _emit_table_loop(table_hbm, idx_hbm, out_hbm, idx_w_base, out_w_base,
                     n_super, idx_v, rows_v, gsem, osem):
    n_chunks = n_super * 4

    def wait_out(b):
        pltpu.make_async_copy(rows_v[b], out_hbm.at[pl.ds(0, CH)], osem[b]).wait()

    def wait_gathers(b):
        pltpu.make_async_copy(out_hbm.at[pl.ds(0, CH)], rows_v[b], gsem[b]).wait()

    @pl.loop(0, n_super // 2)
    def _(gp):
        for ib in (0, 1):
            sb = gp * 2 + ib
            pltpu.sync_copy(idx_hbm.at[pl.ds(idx_w_base + sb * SBR, SBR)],
                            idx_v[ib])
            for c in range(4):
                b = c & 1
                if c >= 2:
                    wait_out(b)
                else:
                    @pl.when(sb >= 1)
                    def _w():
                        wait_out(b)
                for j in range(K):
                    pltpu.async_copy(
                        table_hbm.at[idx_v[ib].at[c * K + j]],
                        rows_v[b].at[pl.ds(j * R, R)],
                        gsem[b])
                prev_out = out_w_base + (sb * 4 + c - 1) * CH

                def _drain(prev_out=prev_out, b=b):
                    wait_gathers(1 - b)
                    pltpu.async_copy(rows_v[1 - b],
                                     out_hbm.at[pl.ds(prev_out, CH)],
                                     osem[1 - b])
                if c >= 1:
                    _drain()
                else:
                    @pl.when(sb >= 1)
                    def _d():
                        _drain()
    wait_gathers(1)
    pltpu.async_copy(
        rows_v[1],
        out_hbm.at[pl.ds(out_w_base + (n_chunks - 1) * CH, CH)], osem[1])
    wait_out(0)
    wait_out(1)


def _sc_gather_both(tri_table, tri_idx, int_table, int_idx):
    n_tri, n_int = tri_idx.shape[0], int_idx.shape[0]
    tri_pw, int_pw = n_tri // NW, n_int // NW
    tri_ns, int_ns = tri_pw // SBR, int_pw // SBR
    assert tri_pw % SBR == 0 and int_pw % SBR == 0
    assert tri_ns % 2 == 0 and int_ns % 2 == 0
    D = tri_table.shape[1]

    mesh = plsc.VectorSubcoreMesh(core_axis_name="c", subcore_axis_name="s")

    @functools.partial(
        pl.kernel,
        mesh=mesh,
        out_type=(jax.ShapeDtypeStruct((n_tri * R, D), jnp.float32),
                  jax.ShapeDtypeStruct((n_int * R, D), jnp.float32)),
        scratch_types=[
            pltpu.VMEM((SBR, R), jnp.int32),
            pltpu.VMEM((SBR, R), jnp.int32),
            pltpu.VMEM((CH, 128), jnp.float32),
            pltpu.VMEM((CH, 128), jnp.float32),
            pltpu.SemaphoreType.DMA,
            pltpu.SemaphoreType.DMA,
            pltpu.SemaphoreType.DMA,
            pltpu.SemaphoreType.DMA,
        ],
    )
    def gather_kernel(tri_t_hbm, tri_i_hbm, int_t_hbm, int_i_hbm,
                      tri_o_hbm, int_o_hbm,
                      idx0, idx1, rows0, rows1, g0, g1, o0, o1):
        wid = lax.axis_index("s") * NC + lax.axis_index("c")
        _emit_table_loop(tri_t_hbm, tri_i_hbm, tri_o_hbm,
                         wid * tri_pw, wid * tri_pw * R, tri_ns,
                         (idx0, idx1), (rows0, rows1), (g0, g1), (o0, o1))
        _emit_table_loop(int_t_hbm, int_i_hbm, int_o_hbm,
                         wid * int_pw, wid * int_pw * R, int_ns,
                         (idx0, idx1), (rows0, rows1), (g0, g1), (o0, o1))

    return gather_kernel(tri_table, tri_idx, int_table, int_idx)


def _tc_mlp(gt, gi, true_l, w1t, b1, w2at, w2bt, b2, w3t, b3):
    B, S, F = gt.shape
    L = true_l
    BB = 32

    def body(gt_ref, gi_ref, w1t_ref, b1_ref, w2at_ref, w2bt_ref, b2_ref,
             w3t_ref, b3_ref, o_ref):
        acc = gt_ref[:, 0, :]
        for s in range(1, S):
            acc = acc + gt_ref[:, s, :]
        x = acc * (1.0 / S)
        t = jnp.dot(x, w1t_ref[...], preferred_element_type=jnp.float32)
        t = jnp.maximum(t + b1_ref[...], 0.0)
        acc2 = gi_ref[:, 0, :]
        for s in range(1, L):
            acc2 = acc2 + gi_ref[:, s, :]
        y = acc2 * (1.0 / L)
        h = (jnp.dot(y, w2at_ref[...], preferred_element_type=jnp.float32)
             + jnp.dot(t, w2bt_ref[...], preferred_element_type=jnp.float32))
        h = jnp.maximum(h + b2_ref[...], 0.0)
        o_ref[...] = (jnp.dot(h, w3t_ref[...], preferred_element_type=jnp.float32)
                      + b3_ref[...])

    return pl.pallas_call(
        body,
        grid=(B // BB,),
        in_specs=[
            pl.BlockSpec((BB, S, F), lambda i: (i, 0, 0)),
            pl.BlockSpec((BB, gi.shape[1], gi.shape[2]), lambda i: (i, 0, 0)),
            pl.BlockSpec(w1t.shape, lambda i: (0, 0)),
            pl.BlockSpec(b1.shape, lambda i: (0, 0)),
            pl.BlockSpec(w2at.shape, lambda i: (0, 0)),
            pl.BlockSpec(w2bt.shape, lambda i: (0, 0)),
            pl.BlockSpec(b2.shape, lambda i: (0, 0)),
            pl.BlockSpec(w3t.shape, lambda i: (0, 0)),
            pl.BlockSpec(b3.shape, lambda i: (0, 0)),
        ],
        out_specs=pl.BlockSpec((BB, w3t.shape[1]), lambda i: (i, 0)),
        out_shape=jax.ShapeDtypeStruct((B, w3t.shape[1]), jnp.float32),
    )(gt, gi, w1t, b1, w2at, w2bt, b2, w3t, b3)


def kernel(trigram_ids, interacted_rate, trigram_table, subreddit_table,
           W1, b1, W2, b2, W3, b3):
    B, S, TRI = trigram_ids.shape
    L = interacted_rate.shape[1]
    EMB = trigram_table.shape[1]

    LP = 64
    tri_idx = trigram_ids.astype(jnp.int32).reshape(-1, 64)
    int_pad = jnp.pad(interacted_rate.astype(jnp.int32),
                      ((0, 0), (0, LP - L)))
    int_idx = int_pad.reshape(-1, 64)

    g_tri, g_int = _sc_gather_both(trigram_table, tri_idx,
                                   subreddit_table, int_idx)

    gt = g_tri.reshape(B, S, TRI * EMB)
    gi = g_int.reshape(B, LP, EMB)

    return _tc_mlp(
        gt, gi, L,
        W1.T, b1.reshape(1, -1),
        W2[:, :EMB].T, W2[:, EMB:].T, b2.reshape(1, -1),
        W3.T, b3.reshape(1, -1),
    )


# speedup vs baseline: 5.0771x; 1.8773x over previous
import functools

import jax
import jax.numpy as jnp
from jax import lax
from jax.experimental import pallas as pl
from jax.experimental.pallas import tpu as pltpu
from jax.experimental.pallas import tpu_sc as plsc

NC = 2
NS = 16
NW = NC * NS


K = 4
R = 64
CH = K * R
SBR = 16


def _emit_table_loop(table_hbm, idx_hbm, out_hbm, idx_w_base, out_w_base,
                     n_super, idx_v, rows_v, gsem, osem):
    n_chunks = n_super * 4

    def wait_out(b):
        pltpu.make_async_copy(rows_v[b], out_hbm.at[pl.ds(0, CH)], osem[b]).wait()

    def wait_gathers(b):
        pltpu.make_async_copy(out_hbm.at[pl.ds(0, CH)], rows_v[b], gsem[b]).wait()

    @pl.loop(0, n_super // 2)
    def _(gp):
        for ib in (0, 1):
            sb = gp * 2 + ib
            pltpu.sync_copy(idx_hbm.at[pl.ds(idx_w_base + sb * SBR, SBR)],
                            idx_v[ib])
            for c in range(4):
                b = c & 1
                if c >= 2:
                    wait_out(b)
                else:
                    @pl.when(sb >= 1)
                    def _w():
                        wait_out(b)
                for j in range(K):
                    pltpu.async_copy(
                        table_hbm.at[idx_v[ib].at[c * K + j]],
                        rows_v[b].at[pl.ds(j * R, R)],
                        gsem[b])
                prev_out = out_w_base + (sb * 4 + c - 1) * CH

                def _drain(prev_out=prev_out, b=b):
                    wait_gathers(1 - b)
                    pltpu.async_copy(rows_v[1 - b],
                                     out_hbm.at[pl.ds(prev_out, CH)],
                                     osem[1 - b])
                if c >= 1:
                    _drain()
                else:
                    @pl.when(sb >= 1)
                    def _d():
                        _drain()
    wait_gathers(1)
    pltpu.async_copy(
        rows_v[1],
        out_hbm.at[pl.ds(out_w_base + (n_chunks - 1) * CH, CH)], osem[1])
    wait_out(0)
    wait_out(1)


def _emit_pooled_loop(table_hbm, idx_hbm, out_hbm, wid, idx_v, rows_v,
                      pooled_v, gsem, osem, isem, n_super, s_len):
    IDXR = 40
    CKI = 5
    CROWS = CKI * R
    OROWS = CROWS // s_len
    per_w_idx = n_super * IDXR
    idx_base = wid * per_w_idx
    out_base = wid * (per_w_idx * R // s_len)

    def fire_chunk(ibuf, c, b):
        for j in range(CKI):
            pltpu.async_copy(
                table_hbm.at[idx_v[ibuf].at[c * CKI + j]],
                rows_v[b].at[pl.ds(j * R, R)],
                gsem[b])

    def wait_gathers(b):
        pltpu.make_async_copy(table_hbm.at[pl.ds(0, CROWS)], rows_v[b],
                              gsem[b]).wait()

    def wait_out(pb):
        pltpu.make_async_copy(pooled_v[pb], out_hbm.at[pl.ds(0, OROWS)],
                              osem[pb]).wait()

    def load_idx_sync(sb, ibuf):
        pltpu.sync_copy(idx_hbm.at[pl.ds(idx_base + sb * IDXR, IDXR)],
                        idx_v[ibuf])

    def load_idx_async(sb, ibuf):
        pltpu.async_copy(idx_hbm.at[pl.ds(idx_base + sb * IDXR, IDXR)],
                         idx_v[ibuf], isem)

    def wait_idx():
        pltpu.make_async_copy(idx_hbm.at[pl.ds(0, IDXR)], idx_v[0],
                              isem).wait()

    def reduce_chunk(b, pb):
        @pl.loop(0, OROWS)
        def _(orow):
            row0 = orow * s_len

            @pl.loop(0, 8)
            def _(g):
                goff = g * 16
                a = rows_v[b][row0, pl.ds(goff, 16)]
                bacc = rows_v[b][row0 + 1, pl.ds(goff, 16)]
                for s in range(2, s_len, 2):
                    a = a + rows_v[b][row0 + s, pl.ds(goff, 16)]
                    bacc = bacc + rows_v[b][row0 + s + 1, pl.ds(goff, 16)]
                pooled_v[pb][orow, pl.ds(goff, 16)] = a + bacc

    load_idx_sync(0, 0)
    fire_chunk(0, 0, 0)
    if n_super > 1:
        load_idx_async(1, 1)

    @pl.loop(0, n_super // 2)
    def _(gp):
        for ib in (0, 1):
            sb = gp * 2 + ib
            for c in range(8):
                b = c & 1
                ci = sb * 8 + c
                wait_gathers(b)
                if c == 0:
                    @pl.when(jnp.logical_and(sb >= 1, sb <= n_super - 2))
                    def _pf():
                        load_idx_async(sb + 1, 1 - ib)
                if c < 7:
                    fire_chunk(ib, c + 1, 1 - b)
                else:
                    @pl.when(sb <= n_super - 2)
                    def _nx():
                        wait_idx()
                        fire_chunk(1 - ib, 0, 1 - b)
                pb = c & 1
                if c >= 2:
                    wait_out(pb)
                else:
                    @pl.when(sb >= 1)
                    def _wo():
                        wait_out(pb)
                reduce_chunk(b, pb)
                pltpu.async_copy(pooled_v[pb],
                                 out_hbm.at[pl.ds(out_base + ci * OROWS,
                                                  OROWS)],
                                 osem[pb])
    wait_out(0)
    wait_out(1)


def _sc_gather_both(tri_table, tri_idx, int_table, int_idx, s_len):
    n_tri, n_int = tri_idx.shape[0], int_idx.shape[0]
    tri_pw, int_pw = n_tri // NW, n_int // NW
    tri_ns = tri_pw // 40
    int_ns = int_pw // SBR
    assert tri_pw % 40 == 0 and int_pw % SBR == 0 and int_ns % 2 == 0
    n_pool = n_tri * R // s_len
    D = tri_table.shape[1]

    mesh = plsc.VectorSubcoreMesh(core_axis_name="c", subcore_axis_name="s")

    @functools.partial(
        pl.kernel,
        mesh=mesh,
        out_type=(jax.ShapeDtypeStruct((n_pool, D), jnp.float32),
                  jax.ShapeDtypeStruct((n_int * R, D), jnp.float32)),
        scratch_types=[
            pltpu.VMEM((40, R), jnp.int32),
            pltpu.VMEM((40, R), jnp.int32),
            pltpu.VMEM((320, 128), jnp.float32),
            pltpu.VMEM((320, 128), jnp.float32),
            pltpu.VMEM((16, 128), jnp.float32),
            pltpu.VMEM((16, 128), jnp.float32),
            pltpu.SemaphoreType.DMA,
            pltpu.SemaphoreType.DMA,
            pltpu.SemaphoreType.DMA,
            pltpu.SemaphoreType.DMA,
            pltpu.SemaphoreType.DMA,
        ],
    )
    def gather_kernel(tri_t_hbm, tri_i_hbm, int_t_hbm, int_i_hbm,
                      tri_o_hbm, int_o_hbm,
                      idx0, idx1, rows0, rows1, pool0, pool1,
                      g0, g1, o0, o1, isem):
        wid = lax.axis_index("s") * NC + lax.axis_index("c")
        _emit_pooled_loop(tri_t_hbm, tri_i_hbm, tri_o_hbm, wid,
                          (idx0, idx1), (rows0, rows1), (pool0, pool1),
                          (g0, g1), (o0, o1), isem, tri_ns, s_len)
        _emit_table_loop(int_t_hbm, int_i_hbm, int_o_hbm,
                         wid * int_pw, wid * int_pw * R, int_ns,
                         (idx0.at[pl.ds(0, SBR)], idx1.at[pl.ds(0, SBR)]),
                         (rows0.at[pl.ds(0, CH)], rows1.at[pl.ds(0, CH)]),
                         (g0, g1), (o0, o1))

    return gather_kernel(tri_table, tri_idx, int_table, int_idx)


def _tc_mlp(xp, gi, true_l, w1t, b1, w2at, w2bt, b2, w3t, b3):
    B, F = xp.shape
    L = true_l
    BB = 256

    def body(xp_ref, gi_ref, w1t_ref, b1_ref, w2at_ref, w2bt_ref, b2_ref,
             w3t_ref, b3_ref, o_ref):
        t = jnp.dot(xp_ref[...], w1t_ref[...],
                    preferred_element_type=jnp.float32)
        t = jnp.maximum(t + b1_ref[...], 0.0)
        acc2 = gi_ref[:, 0, :]
        for s in range(1, L):
            acc2 = acc2 + gi_ref[:, s, :]
        y = acc2 * (1.0 / L)
        h = (jnp.dot(y, w2at_ref[...], preferred_element_type=jnp.float32)
             + jnp.dot(t, w2bt_ref[...], preferred_element_type=jnp.float32))
        h = jnp.maximum(h + b2_ref[...], 0.0)
        o_ref[...] = (jnp.dot(h, w3t_ref[...], preferred_element_type=jnp.float32)
                      + b3_ref[...])

    return pl.pallas_call(
        body,
        grid=(B // BB,),
        in_specs=[
            pl.BlockSpec((BB, F), lambda i: (i, 0)),
            pl.BlockSpec((BB, gi.shape[1], gi.shape[2]), lambda i: (i, 0, 0)),
            pl.BlockSpec(w1t.shape, lambda i: (0, 0)),
            pl.BlockSpec(b1.shape, lambda i: (0, 0)),
            pl.BlockSpec(w2at.shape, lambda i: (0, 0)),
            pl.BlockSpec(w2bt.shape, lambda i: (0, 0)),
            pl.BlockSpec(b2.shape, lambda i: (0, 0)),
            pl.BlockSpec(w3t.shape, lambda i: (0, 0)),
            pl.BlockSpec(b3.shape, lambda i: (0, 0)),
        ],
        out_specs=pl.BlockSpec((BB, w3t.shape[1]), lambda i: (i, 0)),
        out_shape=jax.ShapeDtypeStruct((B, w3t.shape[1]), jnp.float32),
    )(xp, gi, w1t, b1, w2at, w2bt, b2, w3t, b3)


def kernel(trigram_ids, interacted_rate, trigram_table, subreddit_table,
           W1, b1, W2, b2, W3, b3):
    B, S, TRI = trigram_ids.shape
    L = interacted_rate.shape[1]
    EMB = trigram_table.shape[1]

    LP = 64
    tri_idx = (trigram_ids.astype(jnp.int32)
               .transpose(0, 2, 1).reshape(-1, 64))
    int_pad = jnp.pad(interacted_rate.astype(jnp.int32),
                      ((0, 0), (0, LP - L)))
    int_idx = int_pad.reshape(-1, 64)

    g_pool, g_int = _sc_gather_both(trigram_table, tri_idx,
                                    subreddit_table, int_idx, S)

    xp = g_pool.reshape(B, TRI * EMB)
    gi = g_int.reshape(B, LP, EMB)

    return _tc_mlp(
        xp, gi, L,
        W1.T * (1.0 / S), b1.reshape(1, -1),
        W2[:, :EMB].T, W2[:, EMB:].T, b2.reshape(1, -1),
        W3.T, b3.reshape(1, -1),
    )
